# baseline (speedup 1.0000x reference)
import functools

import jax
import jax.numpy as jnp
from jax import lax
from jax.experimental import pallas as pl
from jax.experimental.pallas import tpu as pltpu
from jax.experimental.pallas import tpu_sc as plsc

DIM = 64
R = 8
SCALING = 2.0

NC = 2
NS = 16
NW = NC * NS
IW = 128
NBUF = 4
L = 16
UNROLL = 4


def _sc_fused(xf, weight, lora_a, b2):
    n_idx_rows = xf.shape[0]
    B = n_idx_rows * IW
    rows_pw = n_idx_rows // NW
    n_chunks = rows_pw
    mesh = plsc.VectorSubcoreMesh(core_axis_name="c", subcore_axis_name="s",
                                  num_cores=NC)

    @functools.partial(
        pl.kernel,
        mesh=mesh,
        compiler_params=pltpu.CompilerParams(use_tc_tiling_on_sc=False,
                                             needs_layout_passes=False),
        out_type=jax.ShapeDtypeStruct((B, DIM), jnp.float32),
        scratch_types=[
            pltpu.VMEM((rows_pw, IW), jnp.int32),
            pltpu.VMEM((NBUF, IW, DIM), jnp.float32),
            pltpu.VMEM((NBUF * IW, R), jnp.float32),
            pltpu.VMEM((R, DIM), jnp.float32),
            pltpu.SemaphoreType.DMA,
            pltpu.SemaphoreType.DMA,
            pltpu.SemaphoreType.DMA,
            pltpu.SemaphoreType.DMA,
        ],
    )
    def fused_kernel(xf_hbm, w_hbm, a_hbm, b2_hbm, out_hbm,
                     idx_v, wbuf, abuf, bv, sem_w, sem_a, sem_b, sem_o):
        cid = lax.axis_index("c")
        sid = lax.axis_index("s")
        wid = sid * NC + cid
        r0 = wid * rows_pw
        b0 = r0 * IW
        pltpu.sync_copy(xf_hbm.at[pl.ds(r0, rows_pw)], idx_v)
        pltpu.async_copy(b2_hbm, bv, sem_b).wait()

        bregs = [[bv[k, pl.ds(c * L, L)] for c in range(DIM // L)]
                 for k in range(R)]
        kf = [jnp.full((L,), k, jnp.int32) for k in range(R)]

        def g_copies(c, s):
            return [
                pltpu.make_async_copy(
                    w_hbm.at[idx_v.at[c]], wbuf.at[s], sem_w),
                pltpu.make_async_copy(
                    a_hbm.at[idx_v.at[c]], abuf.at[pl.ds(s * IW, IW)], sem_a),
            ]

        def o_copy(c, s):
            return pltpu.make_async_copy(
                wbuf.at[s], out_hbm.at[pl.ds(b0 + c * IW, IW)], sem_o)

        def compute(s):
            def row_body(it, carry):
                for u in range(UNROLL):
                    r = it * UNROLL + u
                    arow = jnp.full((L,), s * IW, jnp.int32) + r
                    accs = [wbuf[s, r, pl.ds(c * L, L)]
                            for c in range(DIM // L)]
                    for k in range(R):
                        a_s = plsc.load_gather(abuf, [arow, kf[k]])
                        for c in range(DIM // L):
                            accs[c] = accs[c] + a_s * bregs[k][c]
                    for c in range(DIM // L):
                        wbuf[s, r, pl.ds(c * L, L)] = accs[c]
                return carry

            lax.fori_loop(0, IW // UNROLL, row_body, 0)

        LA = NBUF - 1

        def step(c, s1, s3):
            for cp in g_copies(c, s1):
                cp.wait()
            compute(s1)
            o_copy(c, s1).start()

            @pl.when(c + LA < n_chunks)
            def _():
                @pl.when(c >= 1)
                def _():
                    o_copy(c - 1, s3).wait()
                for cp in g_copies(c + LA, s3):
                    cp.start()

        for p in range(LA):
            for cp in g_copies(p, p):
                cp.start()

        def group(t, carry):
            for b in range(NBUF):
                step(t * NBUF + b, b, (b + LA) % NBUF)
            return carry

        full = (n_chunks // NBUF) * NBUF
        lax.fori_loop(0, n_chunks // NBUF, group, 0)
        for c in range(full, n_chunks):
            step(jnp.int32(c), c % NBUF, (c + LA) % NBUF)
        for c in range(n_chunks - NBUF, n_chunks):
            o_copy(c, c % NBUF).wait()

    return fused_kernel(xf, weight, lora_a, b2)


def kernel(x, weight, lora_a, lora_b):
    b2 = lora_b * jnp.float32(SCALING)
    xf = x.reshape(-1).reshape(-1, IW)
    out = _sc_fused(xf, weight, lora_a, b2)
    return out.reshape(x.shape + (DIM,))


# speedup vs baseline: 1.0016x; 1.0016x over previous
import functools

import jax
import jax.numpy as jnp
from jax import lax
from jax.experimental import pallas as pl
from jax.experimental.pallas import tpu as pltpu
from jax.experimental.pallas import tpu_sc as plsc

DIM = 64
R = 8
SCALING = 2.0

NC = 2
NS = 16
NW = NC * NS
IW = 128
NBUF = 3
L = 16
UNROLL = 4


def _sc_fused(xf, weight, lora_a, b2):
    n_idx_rows = xf.shape[0]
    B = n_idx_rows * IW
    rows_pw = n_idx_rows // NW
    n_chunks = rows_pw
    mesh = plsc.VectorSubcoreMesh(core_axis_name="c", subcore_axis_name="s",
                                  num_cores=NC)

    @functools.partial(
        pl.kernel,
        mesh=mesh,
        compiler_params=pltpu.CompilerParams(use_tc_tiling_on_sc=False,
                                             needs_layout_passes=False),
        out_type=jax.ShapeDtypeStruct((B, DIM), jnp.float32),
        scratch_types=[
            pltpu.VMEM((rows_pw, IW), jnp.int32),
            pltpu.VMEM((NBUF, IW, DIM), jnp.float32),
            pltpu.VMEM((NBUF * IW, R), jnp.float32),
            pltpu.VMEM((R, DIM), jnp.float32),
            pltpu.SemaphoreType.DMA,
            pltpu.SemaphoreType.DMA,
            pltpu.SemaphoreType.DMA,
            pltpu.SemaphoreType.DMA,
        ],
    )
    def fused_kernel(xf_hbm, w_hbm, a_hbm, b2_hbm, out_hbm,
                     idx_v, wbuf, abuf, bv, sem_w, sem_a, sem_b, sem_o):
        cid = lax.axis_index("c")
        sid = lax.axis_index("s")
        wid = sid * NC + cid
        r0 = wid * rows_pw
        b0 = r0 * IW
        pltpu.sync_copy(xf_hbm.at[pl.ds(r0, rows_pw)], idx_v)
        pltpu.async_copy(b2_hbm, bv, sem_b).wait()

        bregs = [[bv[k, pl.ds(c * L, L)] for c in range(DIM // L)]
                 for k in range(R)]
        kf = [jnp.full((L,), k, jnp.int32) for k in range(R)]

        def g_copies(c, s):
            return [
                pltpu.make_async_copy(
                    w_hbm.at[idx_v.at[c]], wbuf.at[s], sem_w),
                pltpu.make_async_copy(
                    a_hbm.at[idx_v.at[c]], abuf.at[pl.ds(s * IW, IW)], sem_a),
            ]

        def o_copy(c, s):
            return pltpu.make_async_copy(
                wbuf.at[s], out_hbm.at[pl.ds(b0 + c * IW, IW)], sem_o)

        def compute(s):
            def row_body(it, carry):
                for u in range(UNROLL):
                    r = it * UNROLL + u
                    arow = jnp.full((L,), s * IW, jnp.int32) + r
                    accs = [wbuf[s, r, pl.ds(c * L, L)]
                            for c in range(DIM // L)]
                    for k in range(R):
                        a_s = plsc.load_gather(abuf, [arow, kf[k]])
                        for c in range(DIM // L):
                            accs[c] = accs[c] + a_s * bregs[k][c]
                    for c in range(DIM // L):
                        wbuf[s, r, pl.ds(c * L, L)] = accs[c]
                return carry

            lax.fori_loop(0, IW // UNROLL, row_body, 0)

        LA = NBUF - 1

        def step(c, s1, s3):
            for cp in g_copies(c, s1):
                cp.wait()
            compute(s1)
            o_copy(c, s1).start()

            @pl.when(c + LA < n_chunks)
            def _():
                @pl.when(c >= 1)
                def _():
                    o_copy(c - 1, s3).wait()
                for cp in g_copies(c + LA, s3):
                    cp.start()

        for p in range(LA):
            for cp in g_copies(p, p):
                cp.start()

        def group(t, carry):
            for b in range(NBUF):
                step(t * NBUF + b, b, (b + LA) % NBUF)
            return carry

        full = (n_chunks // NBUF) * NBUF
        lax.fori_loop(0, n_chunks // NBUF, group, 0)
        for c in range(full, n_chunks):
            step(jnp.int32(c), c % NBUF, (c + LA) % NBUF)
        for c in range(n_chunks - NBUF, n_chunks):
            o_copy(c, c % NBUF).wait()

    return fused_kernel(xf, weight, lora_a, b2)


def kernel(x, weight, lora_a, lora_b):
    b2 = lora_b * jnp.float32(SCALING)
    xf = x.reshape(-1).reshape(-1, IW)
    out = _sc_fused(xf, weight, lora_a, b2)
    return out.reshape(x.shape + (DIM,))
